# baseline (speedup 1.0000x reference)
import functools

import jax
import jax.numpy as jnp
from jax import lax
from jax.experimental import pallas as pl
from jax.experimental.pallas import tpu as pltpu
from jax.experimental.pallas import tpu_sc as plsc

B = 16384
D = 32
SUPW = 128
NC = 2
NS = 16
NW = NC * NS
BPW = B // NW
CHUNK = 128
NCH = BPW // CHUNK

_mesh = plsc.VectorSubcoreMesh(core_axis_name="c", subcore_axis_name="s")

_DNUMS = lax.GatherDimensionNumbers(
    offset_dims=(), collapsed_slice_dims=(0,), start_index_map=(0,))


def _shuffle(v, perm):
    return lax.gather(v, perm[:, None], _DNUMS, slice_sizes=(1,),
                      mode=lax.GatherScatterMode.PROMISE_IN_BOUNDS)


@functools.partial(
    pl.kernel,
    mesh=_mesh,
    out_type=jax.ShapeDtypeStruct((B,), jnp.float32),
    scratch_types=[
        pltpu.VMEM((BPW,), jnp.int32),
        pltpu.VMEM((BPW,), jnp.int32),
        pltpu.VMEM((BPW,), jnp.int32),
        pltpu.VMEM((BPW,), jnp.int32),
        pltpu.VMEM((BPW,), jnp.int32),
        pltpu.VMEM((BPW,), jnp.int32),
        pltpu.VMEM((CHUNK, SUPW), jnp.float32),
        pltpu.VMEM((CHUNK, SUPW), jnp.float32),
        pltpu.VMEM((CHUNK, SUPW), jnp.float32),
        pltpu.VMEM((BPW,), jnp.float32),
        pltpu.SemaphoreType.DMA,
    ],
)
def _bpr_sc(ui_hbm, pi_hbm, ni_hbm, user_hbm, item_hbm, out_hbm,
            ui_v, pi_v, ni_v, us_v, ps_v, ns_v, u_b, p_b, n_b, o_v, sem):
    wid = lax.axis_index("s") * NC + lax.axis_index("c")
    base = wid * BPW

    pltpu.sync_copy(ui_hbm.at[pl.ds(base, BPW)], ui_v)
    pltpu.sync_copy(pi_hbm.at[pl.ds(base, BPW)], pi_v)
    pltpu.sync_copy(ni_hbm.at[pl.ds(base, BPW)], ni_v)

    def mk_super(k, carry):
        s = pl.ds(k * 16, 16)
        us_v[s] = ui_v[s] >> 2
        ps_v[s] = pi_v[s] >> 2
        ns_v[s] = ni_v[s] >> 2
        return carry

    lax.fori_loop(0, BPW // 16, mk_super, 0)

    lane = lax.iota(jnp.int32, 16)
    perms = [lane ^ s for s in (8, 4, 2, 1)]

    for c in range(NCH):
        sl = pl.ds(c * CHUNK, CHUNK)
        cps = [pltpu.async_copy(user_hbm.at[us_v.at[sl]], u_b, sem),
               pltpu.async_copy(item_hbm.at[ps_v.at[sl]], p_b, sem),
               pltpu.async_copy(item_hbm.at[ns_v.at[sl]], n_b, sem)]
        for cp in cps:
            cp.wait()

        def group(g, carry, c=c):
            gbase = g * 16
            asl = pl.ds(c * CHUNK + gbase, 16)
            uoffs = (ui_v[asl] & 3) * D
            poffs = (pi_v[asl] & 3) * D
            noffs = (ni_v[asl] & 3) * D
            scores = jnp.zeros((16,), jnp.float32)
            for j in range(16):
                i = gbase + j
                uo = uoffs[j]
                po = poffs[j]
                no = noffs[j]
                u0 = u_b[i, pl.ds(uo, 16)]
                u1 = u_b[i, pl.ds(uo + 16, 16)]
                p0 = p_b[i, pl.ds(po, 16)]
                p1 = p_b[i, pl.ds(po + 16, 16)]
                n0 = n_b[i, pl.ds(no, 16)]
                n1 = n_b[i, pl.ds(no + 16, 16)]
                acc = u0 * (p0 - n0) + u1 * (p1 - n1)
                for perm in perms:
                    acc = acc + _shuffle(acc, perm)
                scores = jnp.where(lane == j, acc, scores)
            o_v[pl.ds(c * CHUNK + gbase, 16)] = scores
            return carry

        lax.fori_loop(0, CHUNK // 16, group, 0)

    pltpu.sync_copy(o_v, out_hbm.at[pl.ds(base, BPW)])


def kernel(x, user_emb, item_emb):
    xi = x.astype(jnp.int32)
    ur = user_emb.reshape(user_emb.shape[0] // 4, SUPW)
    ir = item_emb.reshape(item_emb.shape[0] // 4, SUPW)
    return _bpr_sc(xi[:, 0], xi[:, 1], xi[:, 2], ur, ir)


# speedup vs baseline: 1.0106x; 1.0106x over previous
import functools

import jax
import jax.numpy as jnp
from jax import lax
from jax.experimental import pallas as pl
from jax.experimental.pallas import tpu as pltpu
from jax.experimental.pallas import tpu_sc as plsc

B = 16384
D = 32
NC = 2
NS = 16
NW = NC * NS
BPW = B // NW
CHUNK = 128
NCH = BPW // CHUNK

_mesh = plsc.VectorSubcoreMesh(core_axis_name="c", subcore_axis_name="s")

_DNUMS = lax.GatherDimensionNumbers(
    offset_dims=(), collapsed_slice_dims=(0,), start_index_map=(0,))


def _shuffle(v, perm):
    return lax.gather(v, perm[:, None], _DNUMS, slice_sizes=(1,),
                      mode=lax.GatherScatterMode.PROMISE_IN_BOUNDS)


@functools.partial(
    pl.kernel,
    mesh=_mesh,
    compiler_params=pltpu.CompilerParams(use_tc_tiling_on_sc=False),
    out_type=jax.ShapeDtypeStruct((B,), jnp.float32),
    scratch_types=[
        pltpu.VMEM((BPW,), jnp.int32),
        pltpu.VMEM((BPW,), jnp.int32),
        pltpu.VMEM((BPW,), jnp.int32),
        pltpu.VMEM((BPW, D), jnp.float32),
        pltpu.VMEM((BPW, D), jnp.float32),
        pltpu.VMEM((BPW, D), jnp.float32),
        pltpu.VMEM((BPW,), jnp.float32),
        pltpu.SemaphoreType.DMA,
    ],
)
def _bpr_sc(ui_hbm, pi_hbm, ni_hbm, user_hbm, item_hbm, out_hbm,
            ui_v, pi_v, ni_v, u_v, p_v, n_v, o_v, sem):
    wid = lax.axis_index("s") * NC + lax.axis_index("c")
    base = wid * BPW

    pltpu.sync_copy(ui_hbm.at[pl.ds(base, BPW)], ui_v)
    pltpu.sync_copy(pi_hbm.at[pl.ds(base, BPW)], pi_v)
    pltpu.sync_copy(ni_hbm.at[pl.ds(base, BPW)], ni_v)

    copies = []
    for c in range(NCH):
        sl = pl.ds(c * CHUNK, CHUNK)
        copies.append(pltpu.async_copy(user_hbm.at[ui_v.at[sl]], u_v.at[sl], sem))
        copies.append(pltpu.async_copy(item_hbm.at[pi_v.at[sl]], p_v.at[sl], sem))
        copies.append(pltpu.async_copy(item_hbm.at[ni_v.at[sl]], n_v.at[sl], sem))
    for cp in copies:
        cp.wait()

    lane = lax.iota(jnp.int32, 16)
    perms = [lane ^ s for s in (8, 4, 2, 1)]

    def group(g, carry):
        gbase = g * 16
        scores = jnp.zeros((16,), jnp.float32)
        for j in range(16):
            i = gbase + j
            u0 = u_v[i, pl.ds(0, 16)]
            u1 = u_v[i, pl.ds(16, 16)]
            p0 = p_v[i, pl.ds(0, 16)]
            p1 = p_v[i, pl.ds(16, 16)]
            n0 = n_v[i, pl.ds(0, 16)]
            n1 = n_v[i, pl.ds(16, 16)]
            acc = u0 * (p0 - n0) + u1 * (p1 - n1)
            for perm in perms:
                acc = acc + _shuffle(acc, perm)
            scores = jnp.where(lane == j, acc, scores)
        o_v[pl.ds(gbase, 16)] = scores
        return carry

    lax.fori_loop(0, BPW // 16, group, 0)

    pltpu.sync_copy(o_v, out_hbm.at[pl.ds(base, BPW)])


def kernel(x, user_emb, item_emb):
    xi = x.astype(jnp.int32)
    return _bpr_sc(xi[:, 0], xi[:, 1], xi[:, 2], user_emb, item_emb)
